# baseline (speedup 1.0000x reference)
import functools

import jax
import jax.numpy as jnp
from jax import lax
from jax.experimental import pallas as pl
from jax.experimental.pallas import tpu as pltpu
from jax.experimental.pallas import tpu_sc as plsc

_VOCAB = 30522
_HIDDEN = 768
_SEQ = 128
_BATCH = 1024
_EPS = 1e-12
_NR_ITERS = 8

_RW = 256
_PW = _HIDDEN // 3
_NW = 32
_CH = 128

_NCK = 4
_CB = _BATCH // _NCK
_CTOK = _CB * _SEQ
_TPW = _CTOK // _NW
_NCH = _TPW // _CH

_BB = 32
_NBC = _CB // _BB


def _sc_gather(ids3, table_i32):
    info = plsc.get_sparse_core_info()
    nc = info.num_cores

    mesh = plsc.VectorSubcoreMesh(core_axis_name="c", subcore_axis_name="s")

    @functools.partial(
        pl.kernel,
        mesh=mesh,
        out_type=jax.ShapeDtypeStruct((_CTOK, _RW), jnp.int32),
        scratch_types=[
            pltpu.VMEM((_NCH, _CH), jnp.int32),
            pltpu.VMEM((2, _CH, _RW), jnp.int32),
            pltpu.SemaphoreType.DMA,
            pltpu.SemaphoreType.DMA,
        ],
    )
    def gk(ids_hbm, tab_hbm, out_hbm, idx_v, rows_v, gsem, ssem):
        wid = lax.axis_index("s") * nc + lax.axis_index("c")
        base = wid * _TPW
        pltpu.sync_copy(ids_hbm.at[wid], idx_v)

        pltpu.async_copy(tab_hbm.at[idx_v.at[0]], rows_v.at[0], gsem)

        def body(c, carry):
            buf = lax.rem(c, 2)

            @pl.when(c + 1 < _NCH)
            def _prefetch():
                pltpu.async_copy(
                    tab_hbm.at[idx_v.at[c + 1]], rows_v.at[1 - buf], gsem
                )

            pltpu.make_async_copy(
                tab_hbm.at[idx_v.at[c]], rows_v.at[buf], gsem
            ).wait()
            pltpu.async_copy(
                rows_v.at[buf], out_hbm.at[pl.ds(base + c * _CH, _CH)], ssem
            ).wait()
            return carry

        lax.fori_loop(0, _NCH, body, 0)

    return gk(ids3, table_i32)


def _ln_math(ws_ref, g_ref, tt_ref, ptc0_ref, dt_ref, lnw_ref, lnb_ref,
             out_ref, svar_ref, sr_ref):
    q = g_ref[...]
    ws = ws_ref[0]
    ttf = tt_ref[...].astype(jnp.float32)[:, :, None]
    pieces = []
    for m in range(3):
        b = lax.shift_right_arithmetic(
            lax.shift_left(q, 24 - 8 * m), 24
        ).astype(jnp.float32)
        p0 = ptc0_ref[:, m * _PW:(m + 1) * _PW][None]
        dt = dt_ref[:, m * _PW:(m + 1) * _PW][None]
        em = (b + p0) + ttf * dt
        pieces.append(em)
    e = jnp.concatenate(pieces, axis=2)
    mean = jnp.sum(e, axis=2, keepdims=True) * (1.0 / _HIDDEN)
    var = jnp.sum(e * e, axis=2, keepdims=True) * (1.0 / _HIDDEN) - mean * mean
    svar_ref[...] = var.reshape(_BB, _SEQ)
    s = svar_ref[...] * (ws * ws)
    x = jnp.where(s > 1.0, s * 0.5, jnp.ones_like(s))
    for _ in range(_NR_ITERS):
        x = 0.5 * (x + s / (x + 1e-9))
    sr_ref[...] = 1.0 / (x + _EPS)
    r = sr_ref[...].reshape(_BB, _SEQ, 1)
    out_ref[...] = ((e - mean) * r) * lnw_ref[...][None] + lnb_ref[...][None]


def _ln_first(ws_ref, g_ref, tt_ref, ptc0_ref, dt_ref, lnw_ref, lnb_ref,
              out_ref, svar_ref, sr_ref):
    _ln_math(ws_ref, g_ref, tt_ref, ptc0_ref, dt_ref, lnw_ref, lnb_ref,
             out_ref, svar_ref, sr_ref)


def _ln_chained(ws_ref, g_ref, tt_ref, ptc0_ref, dt_ref, lnw_ref, lnb_ref,
                prev_ref, out_ref, svar_ref, sr_ref):
    del prev_ref
    _ln_math(ws_ref, g_ref, tt_ref, ptc0_ref, dt_ref, lnw_ref, lnb_ref,
             out_ref, svar_ref, sr_ref)


_VB = 512
_NVB = 60
_VPAD = _VB * _NVB


def _build_body(w_ref, out_ref):
    x = w_ref[...].astype(jnp.int32)
    p0 = x[:, 0 * _PW:1 * _PW] & 255
    p1 = x[:, 1 * _PW:2 * _PW] & 255
    p2 = x[:, 2 * _PW:3 * _PW] & 255
    out_ref[...] = p0 | (p1 << 8) | (p2 << 16)


def kernel(input_ids, token_type_ids, word_table, word_scale, pos_table,
           pos_scale, type_table, type_scale, ln_weight, ln_bias):
    ids = input_ids.astype(jnp.int32)

    ext = pl.pallas_call(
        _build_body,
        grid=(_NVB,),
        in_specs=[pl.BlockSpec((_VB, _HIDDEN), lambda i: (i, 0))],
        out_specs=pl.BlockSpec((_VB, _RW), lambda i: (i, 0)),
        out_shape=jax.ShapeDtypeStruct((_VPAD, _RW), jnp.int32),
    )(word_table)

    tt = token_type_ids.astype(jnp.int32)

    posf = pos_table[:_SEQ].astype(jnp.float32) * pos_scale
    t0 = type_table[0].astype(jnp.float32) * type_scale
    t1 = type_table[1].astype(jnp.float32) * type_scale
    ptc0 = (posf + t0[None, :]) / word_scale
    dt = ((t1 - t0) / word_scale)[None, :]
    lnw = (ln_weight * word_scale)[None, :]
    lnb = ln_bias[None, :]
    ws1 = word_scale.reshape(1)

    out_shape = jax.ShapeDtypeStruct((_BATCH, _SEQ, _HIDDEN), jnp.float32)
    gath = [
        _sc_gather(
            lax.slice_in_dim(ids, c * _CB, (c + 1) * _CB).reshape(
                _NW, _NCH, _CH
            ),
            ext,
        ).reshape(_CB, _SEQ, _RW)
        for c in range(_NCK)
    ]

    out = None
    for c in range(_NCK):
        specs = [
            pl.BlockSpec(memory_space=pltpu.SMEM),
            pl.BlockSpec((_BB, _SEQ, _RW), lambda i: (i, 0, 0)),
            pl.BlockSpec((_BB, _SEQ), lambda i: (i, 0)),
            pl.BlockSpec((_SEQ, _HIDDEN), lambda i: (0, 0)),
            pl.BlockSpec((1, _HIDDEN), lambda i: (0, 0)),
            pl.BlockSpec((1, _HIDDEN), lambda i: (0, 0)),
            pl.BlockSpec((1, _HIDDEN), lambda i: (0, 0)),
        ]
        tt_c = lax.slice_in_dim(tt, c * _CB, (c + 1) * _CB)
        args = [ws1, gath[c], tt_c, ptc0, dt, lnw, lnb]
        if out is None:
            body = _ln_first
            aliases = {}
        else:
            body = _ln_chained
            specs.append(pl.BlockSpec(memory_space=pl.ANY))
            args.append(out)
            aliases = {7: 0}
        out = pl.pallas_call(
            body,
            grid=(_NBC,),
            in_specs=specs,
            out_specs=pl.BlockSpec(
                (_BB, _SEQ, _HIDDEN), lambda i, c_=c: (c_ * _NBC + i, 0, 0)
            ),
            out_shape=out_shape,
            scratch_shapes=[
                pltpu.VMEM((_BB, _SEQ), jnp.float32),
                pltpu.VMEM((_BB, _SEQ), jnp.float32),
            ],
            input_output_aliases=aliases,
        )(*args)

    return out


# speedup vs baseline: 1.0059x; 1.0059x over previous
import functools

import jax
import jax.numpy as jnp
from jax import lax
from jax.experimental import pallas as pl
from jax.experimental.pallas import tpu as pltpu
from jax.experimental.pallas import tpu_sc as plsc

_VOCAB = 30522
_HIDDEN = 768
_SEQ = 128
_BATCH = 1024
_EPS = 1e-12
_NR_ITERS = 8

_RW = 256
_PW = _HIDDEN // 3
_NW = 32
_CH = 128

_NCK = 2
_CB = _BATCH // _NCK
_CTOK = _CB * _SEQ
_TPW = _CTOK // _NW
_NCH = _TPW // _CH

_BB = 32
_NBC = _CB // _BB


def _sc_gather(ids3, table_i32):
    info = plsc.get_sparse_core_info()
    nc = info.num_cores

    mesh = plsc.VectorSubcoreMesh(core_axis_name="c", subcore_axis_name="s")

    @functools.partial(
        pl.kernel,
        mesh=mesh,
        out_type=jax.ShapeDtypeStruct((_CTOK, _RW), jnp.int32),
        scratch_types=[
            pltpu.VMEM((_NCH, _CH), jnp.int32),
            pltpu.VMEM((2, _CH, _RW), jnp.int32),
            pltpu.SemaphoreType.DMA,
            pltpu.SemaphoreType.DMA,
        ],
    )
    def gk(ids_hbm, tab_hbm, out_hbm, idx_v, rows_v, gsem, ssem):
        wid = lax.axis_index("s") * nc + lax.axis_index("c")
        base = wid * _TPW
        pltpu.sync_copy(ids_hbm.at[wid], idx_v)

        pltpu.async_copy(tab_hbm.at[idx_v.at[0]], rows_v.at[0], gsem)

        def body(c, carry):
            buf = lax.rem(c, 2)

            @pl.when(c + 1 < _NCH)
            def _prefetch():
                pltpu.async_copy(
                    tab_hbm.at[idx_v.at[c + 1]], rows_v.at[1 - buf], gsem
                )

            pltpu.make_async_copy(
                tab_hbm.at[idx_v.at[c]], rows_v.at[buf], gsem
            ).wait()
            pltpu.async_copy(
                rows_v.at[buf], out_hbm.at[pl.ds(base + c * _CH, _CH)], ssem
            ).wait()
            return carry

        lax.fori_loop(0, _NCH, body, 0)

    return gk(ids3, table_i32)


def _ln_math(ws_ref, g_ref, tt_ref, ptc0_ref, dt_ref, lnw_ref, lnb_ref,
             out_ref, svar_ref, sr_ref):
    q = g_ref[...]
    ws = ws_ref[0]
    ttf = tt_ref[...].astype(jnp.float32)[:, :, None]
    pieces = []
    for m in range(3):
        b = lax.shift_right_arithmetic(
            lax.shift_left(q, 24 - 8 * m), 24
        ).astype(jnp.float32)
        p0 = ptc0_ref[:, m * _PW:(m + 1) * _PW][None]
        dt = dt_ref[:, m * _PW:(m + 1) * _PW][None]
        em = (b + p0) + ttf * dt
        pieces.append(em)
    e = jnp.concatenate(pieces, axis=2)
    mean = jnp.sum(e, axis=2, keepdims=True) * (1.0 / _HIDDEN)
    var = jnp.sum(e * e, axis=2, keepdims=True) * (1.0 / _HIDDEN) - mean * mean
    svar_ref[...] = var.reshape(_BB, _SEQ)
    s = svar_ref[...] * (ws * ws)
    x = jnp.where(s > 1.0, s * 0.5, jnp.ones_like(s))
    for _ in range(_NR_ITERS):
        x = 0.5 * (x + s / (x + 1e-9))
    sr_ref[...] = 1.0 / (x + _EPS)
    r = sr_ref[...].reshape(_BB, _SEQ, 1)
    out_ref[...] = ((e - mean) * r) * lnw_ref[...][None] + lnb_ref[...][None]


def _ln_first(ws_ref, g_ref, tt_ref, ptc0_ref, dt_ref, lnw_ref, lnb_ref,
              out_ref, svar_ref, sr_ref):
    _ln_math(ws_ref, g_ref, tt_ref, ptc0_ref, dt_ref, lnw_ref, lnb_ref,
             out_ref, svar_ref, sr_ref)


def _ln_chained(ws_ref, g_ref, tt_ref, ptc0_ref, dt_ref, lnw_ref, lnb_ref,
                prev_ref, out_ref, svar_ref, sr_ref):
    del prev_ref
    _ln_math(ws_ref, g_ref, tt_ref, ptc0_ref, dt_ref, lnw_ref, lnb_ref,
             out_ref, svar_ref, sr_ref)


_VB = 512
_NVB = 60
_VPAD = _VB * _NVB


def _build_body(w_ref, out_ref):
    x = w_ref[...].astype(jnp.int32)
    p0 = x[:, 0 * _PW:1 * _PW] & 255
    p1 = x[:, 1 * _PW:2 * _PW] & 255
    p2 = x[:, 2 * _PW:3 * _PW] & 255
    out_ref[...] = p0 | (p1 << 8) | (p2 << 16)


def kernel(input_ids, token_type_ids, word_table, word_scale, pos_table,
           pos_scale, type_table, type_scale, ln_weight, ln_bias):
    ids = input_ids.astype(jnp.int32)

    ext = pl.pallas_call(
        _build_body,
        grid=(_NVB,),
        in_specs=[pl.BlockSpec((_VB, _HIDDEN), lambda i: (i, 0))],
        out_specs=pl.BlockSpec((_VB, _RW), lambda i: (i, 0)),
        out_shape=jax.ShapeDtypeStruct((_VPAD, _RW), jnp.int32),
    )(word_table)

    tt = token_type_ids.astype(jnp.int32)

    posf = pos_table[:_SEQ].astype(jnp.float32) * pos_scale
    t0 = type_table[0].astype(jnp.float32) * type_scale
    t1 = type_table[1].astype(jnp.float32) * type_scale
    ptc0 = (posf + t0[None, :]) / word_scale
    dt = ((t1 - t0) / word_scale)[None, :]
    lnw = (ln_weight * word_scale)[None, :]
    lnb = ln_bias[None, :]
    ws1 = word_scale.reshape(1)

    out_shape = jax.ShapeDtypeStruct((_BATCH, _SEQ, _HIDDEN), jnp.float32)
    gath = [
        _sc_gather(
            lax.slice_in_dim(ids, c * _CB, (c + 1) * _CB).reshape(
                _NW, _NCH, _CH
            ),
            ext,
        ).reshape(_CB, _SEQ, _RW)
        for c in range(_NCK)
    ]

    out = None
    for c in range(_NCK):
        specs = [
            pl.BlockSpec(memory_space=pltpu.SMEM),
            pl.BlockSpec((_BB, _SEQ, _RW), lambda i: (i, 0, 0)),
            pl.BlockSpec((_BB, _SEQ), lambda i: (i, 0)),
            pl.BlockSpec((_SEQ, _HIDDEN), lambda i: (0, 0)),
            pl.BlockSpec((1, _HIDDEN), lambda i: (0, 0)),
            pl.BlockSpec((1, _HIDDEN), lambda i: (0, 0)),
            pl.BlockSpec((1, _HIDDEN), lambda i: (0, 0)),
        ]
        tt_c = lax.slice_in_dim(tt, c * _CB, (c + 1) * _CB)
        args = [ws1, gath[c], tt_c, ptc0, dt, lnw, lnb]
        if out is None:
            body = _ln_first
            aliases = {}
        else:
            body = _ln_chained
            specs.append(pl.BlockSpec(memory_space=pl.ANY))
            args.append(out)
            aliases = {7: 0}
        out = pl.pallas_call(
            body,
            grid=(_NBC,),
            in_specs=specs,
            out_specs=pl.BlockSpec(
                (_BB, _SEQ, _HIDDEN), lambda i, c_=c: (c_ * _NBC + i, 0, 0)
            ),
            out_shape=out_shape,
            scratch_shapes=[
                pltpu.VMEM((_BB, _SEQ), jnp.float32),
                pltpu.VMEM((_BB, _SEQ), jnp.float32),
            ],
            input_output_aliases=aliases,
        )(*args)

    return out


# speedup vs baseline: 1.0119x; 1.0059x over previous
import functools

import jax
import jax.numpy as jnp
from jax import lax
from jax.experimental import pallas as pl
from jax.experimental.pallas import tpu as pltpu
from jax.experimental.pallas import tpu_sc as plsc

_VOCAB = 30522
_HIDDEN = 768
_SEQ = 128
_BATCH = 1024
_EPS = 1e-12
_NR_ITERS = 8

_RW = 256
_PW = _HIDDEN // 3
_TOK = _BATCH * _SEQ
_NW = 32
_TPW = _TOK // _NW
_CH = 128
_NCH = _TPW // _CH

_BB = 32
_NB = _BATCH // _BB


def _sc_gather(ids3, table_i32):
    info = plsc.get_sparse_core_info()
    nc = info.num_cores

    mesh = plsc.VectorSubcoreMesh(core_axis_name="c", subcore_axis_name="s")

    @functools.partial(
        pl.kernel,
        mesh=mesh,
        out_type=jax.ShapeDtypeStruct((_TOK, _RW), jnp.int32),
        scratch_types=[
            pltpu.VMEM((_NCH, _CH), jnp.int32),
            pltpu.VMEM((2, _CH, _RW), jnp.int32),
            pltpu.SemaphoreType.DMA,
            pltpu.SemaphoreType.DMA,
        ],
    )
    def gk(ids_hbm, tab_hbm, out_hbm, idx_v, rows_v, gsem, ssem):
        wid = lax.axis_index("s") * nc + lax.axis_index("c")
        base = wid * _TPW
        pltpu.sync_copy(ids_hbm.at[wid], idx_v)

        pltpu.async_copy(tab_hbm.at[idx_v.at[0]], rows_v.at[0], gsem)

        def body(c, carry):
            buf = lax.rem(c, 2)

            @pl.when(c + 1 < _NCH)
            def _prefetch():
                pltpu.async_copy(
                    tab_hbm.at[idx_v.at[c + 1]], rows_v.at[1 - buf], gsem
                )

            pltpu.make_async_copy(
                tab_hbm.at[idx_v.at[c]], rows_v.at[buf], gsem
            ).wait()
            pltpu.async_copy(
                rows_v.at[buf], out_hbm.at[pl.ds(base + c * _CH, _CH)], ssem
            ).wait()
            return carry

        lax.fori_loop(0, _NCH, body, 0)

    return gk(ids3, table_i32)


def _ln_body(ws_ref, g_ref, tt_ref, ptc0_ref, dt_ref, lnw_ref, lnb_ref,
             out_ref, svar_ref, sr_ref):
    q = g_ref[...]
    ws = ws_ref[0]
    ttf = tt_ref[...].astype(jnp.float32)[:, :, None]
    pieces = []
    for m in range(3):
        b = lax.shift_right_arithmetic(
            lax.shift_left(q, 24 - 8 * m), 24
        ).astype(jnp.float32)
        p0 = ptc0_ref[:, m * _PW:(m + 1) * _PW][None]
        dt = dt_ref[:, m * _PW:(m + 1) * _PW][None]
        em = (b + p0) + ttf * dt
        pieces.append(em)
    e = jnp.concatenate(pieces, axis=2)
    mean = jnp.sum(e, axis=2, keepdims=True) * (1.0 / _HIDDEN)
    var = jnp.sum(e * e, axis=2, keepdims=True) * (1.0 / _HIDDEN) - mean * mean
    svar_ref[...] = var.reshape(_BB, _SEQ)
    s = svar_ref[...] * (ws * ws)
    x = jnp.where(s > 1.0, s * 0.5, jnp.ones_like(s))
    for _ in range(_NR_ITERS):
        x = 0.5 * (x + s / (x + 1e-9))
    sr_ref[...] = 1.0 / (x + _EPS)
    r = sr_ref[...].reshape(_BB, _SEQ, 1)
    out_ref[...] = ((e - mean) * r) * lnw_ref[...][None] + lnb_ref[...][None]


_VB = 512
_NVB = 60
_VPAD = _VB * _NVB


def _build_body(w_ref, out_ref):
    x = w_ref[...].astype(jnp.int32)
    p0 = x[:, 0 * _PW:1 * _PW] & 255
    p1 = x[:, 1 * _PW:2 * _PW] & 255
    p2 = x[:, 2 * _PW:3 * _PW] & 255
    out_ref[...] = p0 | (p1 << 8) | (p2 << 16)


def kernel(input_ids, token_type_ids, word_table, word_scale, pos_table,
           pos_scale, type_table, type_scale, ln_weight, ln_bias):
    ids_eff = input_ids.astype(jnp.int32).reshape(_NW, _NCH, _CH)

    ext = pl.pallas_call(
        _build_body,
        grid=(_NVB,),
        in_specs=[pl.BlockSpec((_VB, _HIDDEN), lambda i: (i, 0))],
        out_specs=pl.BlockSpec((_VB, _RW), lambda i: (i, 0)),
        out_shape=jax.ShapeDtypeStruct((_VPAD, _RW), jnp.int32),
    )(word_table)

    gathered = _sc_gather(ids_eff, ext)
    g3 = gathered.reshape(_BATCH, _SEQ, _RW)
    tt = token_type_ids.astype(jnp.int32)

    posf = pos_table[:_SEQ].astype(jnp.float32) * pos_scale
    t0 = type_table[0].astype(jnp.float32) * type_scale
    t1 = type_table[1].astype(jnp.float32) * type_scale
    ptc0 = (posf + t0[None, :]) / word_scale
    dt = ((t1 - t0) / word_scale)[None, :]
    lnw = (ln_weight * word_scale)[None, :]
    lnb = ln_bias[None, :]
    ws1 = word_scale.reshape(1)

    out = pl.pallas_call(
        _ln_body,
        grid=(_NB,),
        in_specs=[
            pl.BlockSpec(memory_space=pltpu.SMEM),
            pl.BlockSpec((_BB, _SEQ, _RW), lambda i: (i, 0, 0)),
            pl.BlockSpec((_BB, _SEQ), lambda i: (i, 0)),
            pl.BlockSpec((_SEQ, _HIDDEN), lambda i: (0, 0)),
            pl.BlockSpec((1, _HIDDEN), lambda i: (0, 0)),
            pl.BlockSpec((1, _HIDDEN), lambda i: (0, 0)),
            pl.BlockSpec((1, _HIDDEN), lambda i: (0, 0)),
        ],
        out_specs=pl.BlockSpec((_BB, _SEQ, _HIDDEN), lambda i: (i, 0, 0)),
        out_shape=jax.ShapeDtypeStruct((_BATCH, _SEQ, _HIDDEN), jnp.float32),
        scratch_shapes=[
            pltpu.VMEM((_BB, _SEQ), jnp.float32),
            pltpu.VMEM((_BB, _SEQ), jnp.float32),
        ],
    )(ws1, g3, tt, ptc0, dt, lnw, lnb)

    return out
